# baseline (speedup 1.0000x reference)
import functools

import jax
import jax.numpy as jnp
from jax import lax
from jax.experimental import pallas as pl
from jax.experimental.pallas import tpu as pltpu
from jax.experimental.pallas import tpu_sc as plsc

N = 10000
E = 320000
F = 128
H = 256
AVS = 128
P = 16384

NC = 2
NS = 16
NW = NC * NS
EW = E // NW
B = 128
NBLK = 80
EWP = NBLK * B
CH = 8
NCH = NBLK // CH
NP = 10112
RS = NP // NS
HH = NP // 2
K = 2

_MESH = plsc.VectorSubcoreMesh(core_axis_name="c", subcore_axis_name="s")


def _make_edge_pass(C):
  @functools.partial(
      pl.kernel,
      out_type=jax.ShapeDtypeStruct((2, C, NP, F), jnp.float32),
      mesh=_MESH,
      scratch_types=(
          [pltpu.VMEM((NBLK, B), jnp.int32)]
          + [pltpu.VMEM((CH, B), jnp.int32) for _ in range(2)]
          + [pltpu.VMEM((B, F), jnp.float32) for _ in range(K)]
          + [pltpu.SemaphoreType.DMA for _ in range(2 * K + 2)]
          + [pltpu.VMEM_SHARED((NP, F), jnp.float32)]
      ),
  )
  def edge_kernel(table_hbm, srcoff_hbm, dst_hbm, zeros_hbm, out_hbm,
                  dstbuf, *rest):
    srcb = rest[:2]
    rows = rest[2:2 + K]
    gsem = rest[2 + K:2 + 2 * K]
    ssem = rest[2 + 2 * K:2 + 3 * K]
    isem = rest[2 + 3 * K:2 + 3 * K + 2]
    acc = rest[2 + 3 * K + 2]
    c = lax.axis_index("c")
    s = lax.axis_index("s")
    w = c * NS + s
    rbase = s * RS
    pltpu.sync_copy(dst_hbm.at[w], dstbuf)
    for h in range(C):
      pltpu.sync_copy(zeros_hbm.at[pl.ds(rbase, RS)], acc.at[pl.ds(rbase, RS)])
      pltpu.sync_copy(srcoff_hbm.at[h, w, pl.ds(0, CH)], srcb[0])
      pltpu.async_copy(srcoff_hbm.at[h, w, pl.ds(CH, CH)], srcb[1], isem[1])
      plsc.subcore_barrier()

      for k in range(K):
        pltpu.async_copy(table_hbm.at[srcb[0].at[k]], rows[k], gsem[k])

      for i in range(NBLK):
        k = i % K
        jc = i // CH
        b = i % CH
        pltpu.make_async_copy(table_hbm.at[srcb[0].at[0]], rows[k],
                              gsem[k]).wait()
        if b == CH - 1 and jc + 2 < NCH:
          pltpu.async_copy(srcoff_hbm.at[h, w, pl.ds((jc + 2) * CH, CH)],
                           srcb[jc % 2], isem[jc % 2])
        pltpu.async_copy(rows[k], acc.at[dstbuf.at[i]], ssem[k], add=True)
        i2 = i + K
        if i2 < NBLK:
          if b == CH - 2 and jc + 1 < NCH:
            pltpu.make_async_copy(srcoff_hbm.at[h, w, pl.ds(0, CH)],
                                  srcb[(jc + 1) % 2], isem[(jc + 1) % 2]).wait()
          pltpu.make_async_copy(rows[k], acc.at[dstbuf.at[0]], ssem[k]).wait()
          pltpu.async_copy(table_hbm.at[srcb[(i2 // CH) % 2].at[i2 % CH]],
                           rows[k], gsem[k])
      for k in range(K):
        pltpu.make_async_copy(rows[k], acc.at[dstbuf.at[0]], ssem[k]).wait()
      plsc.subcore_barrier()
      pltpu.sync_copy(acc.at[pl.ds(rbase, RS)],
                      out_hbm.at[c, h, pl.ds(rbase, RS)])
      plsc.subcore_barrier()

  return edge_kernel


_edge_pass_1 = _make_edge_pass(1)
_edge_pass_2 = _make_edge_pass(2)


@functools.partial(
    pl.kernel,
    out_type=jax.ShapeDtypeStruct((3 * NW * NP,), jnp.float32),
    mesh=_MESH,
    compiler_params=pltpu.CompilerParams(needs_layout_passes=False),
    scratch_types=[
        pltpu.VMEM((EWP,), jnp.int32),
        pltpu.VMEM((16 * HH,), jnp.float32),
        pltpu.VMEM((HH,), jnp.float32),
    ],
)
def _deg_kernel(dsts_hbm, out_hbm, dstbuf, hist, res):
  c = lax.axis_index("c")
  s = lax.axis_index("s")
  w = c * NS + s
  lane = lax.iota(jnp.int32, 16)
  zero16 = jnp.zeros((16,), jnp.float32)
  for m in range(3):
    pltpu.sync_copy(dsts_hbm.at[pl.ds(m * NW * EWP + w * EWP, EWP)], dstbuf)
    for half in range(2):

      def zbody(j, carry):
        for r in range(16):
          hist[pl.ds(r * HH + j * 16, 16)] = zero16
        return carry

      lax.fori_loop(0, HH // 16, zbody, 0)

      def sbody(j, carry):
        d16 = dstbuf[pl.ds(j * 16, 16)]
        loc = d16 - half * HH
        mask = (loc >= 0) & (loc < HH)
        val = jnp.where(mask, 1.0, 0.0).astype(jnp.float32)
        locc = jnp.clip(loc, 0, HH - 1) + lane * HH
        cur = plsc.load_gather(hist, [locc])
        plsc.store_scatter(hist, [locc], cur + val)
        return carry

      lax.fori_loop(0, EWP // 16, sbody, 0)

      def rbody(j, carry):
        acc = hist[pl.ds(j * 16, 16)]
        for r in range(1, 16):
          acc = acc + hist[pl.ds(r * HH + j * 16, 16)]
        res[pl.ds(j * 16, 16)] = acc
        return carry

      lax.fori_loop(0, HH // 16, rbody, 0)
      pltpu.sync_copy(res, out_hbm.at[pl.ds(m * NW * NP + w * NP + half * HH, HH)])


PB = P // NW
AB = 128


@functools.partial(
    pl.kernel,
    out_type=jax.ShapeDtypeStruct((2, P, AVS), jnp.float32),
    mesh=_MESH,
    scratch_types=[
        pltpu.VMEM((AB,), jnp.int32),
        pltpu.VMEM((AB, AVS), jnp.float32),
        pltpu.SemaphoreType.DMA,
    ],
)
def _action_gather(final_hbm, a01_hbm, out_hbm, idx_v, rows_v, sem):
  c = lax.axis_index("c")
  s = lax.axis_index("s")
  w = c * NS + s
  base = w * PB
  for t in range(2):

    def body(i, carry):
      off = base + i * AB
      pltpu.sync_copy(a01_hbm.at[pl.ds(t * P + off, AB)], idx_v)
      pltpu.async_copy(final_hbm.at[idx_v], rows_v, sem).wait()
      pltpu.sync_copy(rows_v, out_hbm.at[t, pl.ds(off, AB)])
      return carry

    lax.fori_loop(0, PB // AB, body, 0)


def _prep_body(W2l, W2r, b2, W3l, W3r, b3, WlA, blA, WlB, blB, Wout, bout,
               GL, GR, gb, bfin):
  f32 = jnp.float32
  T = jnp.dot(WlB[...], Wout[...], preferred_element_type=f32)
  Whead = jnp.dot(WlA[...], T, preferred_element_type=f32)
  bhead = (jnp.dot(blA[...], T, preferred_element_type=f32)
           + jnp.dot(blB[...], Wout[...], preferred_element_type=f32)
           + bout[...])
  C3 = jnp.dot(W3l[...], Whead, preferred_element_type=f32)
  D3 = jnp.dot(W3r[...], Whead, preferred_element_type=f32)
  CD = jnp.concatenate([C3, D3], axis=1)
  GL[...] = jnp.dot(W2l[...], CD, preferred_element_type=f32)
  GR[...] = jnp.dot(W2r[...], CD, preferred_element_type=f32)
  gb[...] = jnp.dot(b2[...], CD, preferred_element_type=f32)
  bfin[...] = jnp.dot(b3[...], Whead, preferred_element_type=f32) + bhead


NB = 1000


def _degsum(deg_ref):
  return jnp.maximum(jnp.sum(deg_ref[...], axis=1), 1.0)[:, None]


def _combine1_body(parts, deg_ref, obs, W1l, W1r, b1, x1T):
  agg = parts[0, 0] + parts[1, 0]
  mean = agg / _degsum(deg_ref)
  x1 = (jnp.dot(mean, W1l[...], preferred_element_type=jnp.float32)
        + jnp.dot(obs[...], W1r[...], preferred_element_type=jnp.float32)
        + b1[...])
  x1T[0] = x1[:, :F]
  x1T[1] = x1[:, F:]


def _combine2_body(parts, deg_ref, x1T, GL, GR, gb, y3T, z):
  a0 = parts[0, 0] + parts[1, 0]
  a1 = parts[0, 1] + parts[1, 1]
  mean = jnp.concatenate([a0, a1], axis=1) / _degsum(deg_ref)
  x1 = jnp.concatenate([x1T[0], x1T[1]], axis=1)
  yz = (jnp.dot(mean, GL[...], preferred_element_type=jnp.float32)
        + jnp.dot(x1, GR[...], preferred_element_type=jnp.float32)
        + gb[...])
  y3T[...] = yz[:, :F]
  z[...] = yz[:, F:]


def _combine3_body(parts, deg_ref, z, bfin, final):
  agg = parts[0, 0] + parts[1, 0]
  final[...] = agg / _degsum(deg_ref) + z[...] + bfin[...]


def _dotsoft_body(g, out):
  sv = g[0, :, :AVS // 2]
  dv = g[1, :, AVS // 2:]
  logits = jnp.sum(sv * dv, axis=1)
  m = jnp.max(logits)
  e = jnp.exp(logits - m)
  out[0, :] = e / jnp.sum(e)


def kernel(actions, obs, eic, eid, eit, W1l, W1r, b1, W2l, W2r, b2,
           W3l, W3r, b3, WlA, blA, WlB, blB, Wout, bout):
  i32 = jnp.int32
  f32 = jnp.float32
  s_t, d_t = eit[0].astype(i32), eit[1].astype(i32)
  s_c, d_c = eic[0].astype(i32), eic[1].astype(i32)
  s_d, d_d = eid[0].astype(i32), eid[1].astype(i32)
  a01 = jnp.concatenate([actions[..., 0].reshape(-1).astype(i32),
                         actions[..., 1].reshape(-1).astype(i32)])

  def pad_edges(sa, da):
    s2 = sa.reshape(NW, EW)
    d2 = da.reshape(NW, EW)
    padv = jnp.broadcast_to(jnp.arange(EWP - EW, dtype=i32)[None],
                            (NW, EWP - EW))
    s2p = jnp.concatenate([s2, padv % N], axis=1)
    d2p = jnp.concatenate([d2, padv % (NP - N) + N], axis=1)
    return s2p, d2p

  s_tp, d_tp = pad_edges(s_t, d_t)
  s_cp, d_cp = pad_edges(s_c, d_c)
  s_dp, d_dp = pad_edges(s_d, d_d)
  dsts = jnp.concatenate([d_tp.reshape(-1), d_cp.reshape(-1),
                          d_dp.reshape(-1)])
  zeros = jnp.zeros((NP, F), f32)
  b1r = b1.reshape(1, H)
  b2r = b2.reshape(1, H)
  b3r = b3.reshape(1, H)
  blAr = blA.reshape(1, H)
  blBr = blB.reshape(1, H)
  boutr = bout.reshape(1, AVS)

  full = lambda shp: pl.BlockSpec(shp, lambda i: (0,) * len(shp))

  degs = _deg_kernel(dsts).reshape(3, NW, NP).transpose(0, 2, 1)

  GL, GR, gb, bfin = pl.pallas_call(
      _prep_body,
      grid=(1,),
      in_specs=[full((H, H)), full((H, H)), full((1, H)), full((H, H)),
                full((H, H)), full((1, H)), full((H, H)), full((1, H)),
                full((H, H)), full((1, H)), full((H, AVS)), full((1, AVS))],
      out_specs=[full((H, H)), full((H, H)), full((1, H)), full((1, AVS))],
      out_shape=[jax.ShapeDtypeStruct((H, H), f32),
                 jax.ShapeDtypeStruct((H, H), f32),
                 jax.ShapeDtypeStruct((1, H), f32),
                 jax.ShapeDtypeStruct((1, AVS), f32)],
  )(W2l, W2r, b2r, W3l, W3r, b3r, WlA, blAr, WlB, blBr, Wout, boutr)

  p1 = _edge_pass_1(obs, s_tp.reshape(1, NW, NBLK, B),
                    d_tp.reshape(NW, NBLK, B), zeros)

  x1T = pl.pallas_call(
      _combine1_body,
      grid=(N // NB,),
      in_specs=[
          pl.BlockSpec((2, 1, NB, F), lambda i: (0, 0, i, 0)),
          pl.BlockSpec((NB, NW), lambda i: (i, 0)),
          pl.BlockSpec((NB, F), lambda i: (i, 0)),
          full((F, H)), full((F, H)), full((1, H)),
      ],
      out_specs=pl.BlockSpec((2, NB, F), lambda i: (0, i, 0)),
      out_shape=jax.ShapeDtypeStruct((2, N, F), f32),
  )(p1, degs[0], obs, W1l, W1r, b1r)

  srcoff_c = jnp.stack([s_cp, s_cp + N]).reshape(2, NW, NBLK, B)
  p2 = _edge_pass_2(x1T.reshape(2 * N, F), srcoff_c,
                    d_cp.reshape(NW, NBLK, B), zeros)

  y3T, z = pl.pallas_call(
      _combine2_body,
      grid=(N // NB,),
      in_specs=[
          pl.BlockSpec((2, 2, NB, F), lambda i: (0, 0, i, 0)),
          pl.BlockSpec((NB, NW), lambda i: (i, 0)),
          pl.BlockSpec((2, NB, F), lambda i: (0, i, 0)),
          full((H, H)), full((H, H)), full((1, H)),
      ],
      out_specs=[pl.BlockSpec((NB, F), lambda i: (i, 0)),
                 pl.BlockSpec((NB, AVS), lambda i: (i, 0))],
      out_shape=[jax.ShapeDtypeStruct((N, F), f32),
                 jax.ShapeDtypeStruct((N, AVS), f32)],
  )(p2, degs[1], x1T, GL, GR, gb)

  p3 = _edge_pass_1(y3T, s_dp.reshape(1, NW, NBLK, B),
                    d_dp.reshape(NW, NBLK, B), zeros)

  final = pl.pallas_call(
      _combine3_body,
      grid=(N // NB,),
      in_specs=[
          pl.BlockSpec((2, 1, NB, F), lambda i: (0, 0, i, 0)),
          pl.BlockSpec((NB, NW), lambda i: (i, 0)),
          pl.BlockSpec((NB, AVS), lambda i: (i, 0)),
          full((1, AVS)),
      ],
      out_specs=pl.BlockSpec((NB, AVS), lambda i: (i, 0)),
      out_shape=jax.ShapeDtypeStruct((N, AVS), f32),
  )(p3, degs[2], z, bfin)

  g = _action_gather(final, a01)
  probs = pl.pallas_call(
      _dotsoft_body,
      grid=(1,),
      in_specs=[full((2, P, AVS))],
      out_specs=full((1, P)),
      out_shape=jax.ShapeDtypeStruct((1, P), f32),
  )(g)
  return probs


# speedup vs baseline: 1.1262x; 1.1262x over previous
import functools

import jax
import jax.numpy as jnp
from jax import lax
from jax.experimental import pallas as pl
from jax.experimental.pallas import tpu as pltpu
from jax.experimental.pallas import tpu_sc as plsc

N = 10000
E = 320000
F = 128
H = 256
AVS = 128
P = 16384

NC = 2
NS = 16
NW = NC * NS
EW = E // NW
B = 128
NBLK = 80
EWP = NBLK * B
CH = 8
NCH = NBLK // CH
NP = 10112
RS = NP // NS
HH = NP // 2
K = 2

_MESH = plsc.VectorSubcoreMesh(core_axis_name="c", subcore_axis_name="s")


def _make_edge_pass(C):
  @functools.partial(
      pl.kernel,
      out_type=jax.ShapeDtypeStruct((2, C, NP, F), jnp.float32),
      mesh=_MESH,
      scratch_types=(
          [pltpu.VMEM((NBLK, B), jnp.int32)]
          + [pltpu.VMEM((B,), jnp.int32) for _ in range(K)]
          + [pltpu.VMEM((B, F), jnp.float32) for _ in range(K)]
          + [pltpu.SemaphoreType.DMA for _ in range(3 * K)]
          + [pltpu.VMEM_SHARED((NP, F), jnp.float32)]
      ),
  )
  def edge_kernel(table_hbm, srcoff_hbm, dst_hbm, zeros_hbm, out_hbm,
                  dstbuf, *rest):
    srcb = rest[:K]
    rows = rest[K:2 * K]
    gsem = rest[2 * K:3 * K]
    ssem = rest[3 * K:4 * K]
    isem = rest[4 * K:5 * K]
    acc = rest[5 * K]
    c = lax.axis_index("c")
    s = lax.axis_index("s")
    w = c * NS + s
    rbase = s * RS
    pltpu.sync_copy(dst_hbm.at[w], dstbuf)
    for h in range(C):
      pltpu.sync_copy(zeros_hbm.at[pl.ds(rbase, RS)], acc.at[pl.ds(rbase, RS)])
      plsc.subcore_barrier()

      for k in range(K):
        pltpu.sync_copy(srcoff_hbm.at[h, w, k], srcb[k])
        pltpu.async_copy(table_hbm.at[srcb[k]], rows[k], gsem[k])

      def body(j, carry):
        for k in range(K):
          i = j * K + k

          @pl.when(i < NBLK)
          def _process():
            nxt = i + K
            pltpu.make_async_copy(table_hbm.at[srcb[0]], rows[k],
                                  gsem[k]).wait()

            @pl.when(nxt < NBLK)
            def _prefetch_idx():
              pltpu.async_copy(srcoff_hbm.at[h, w, nxt], srcb[k], isem[k])

            pltpu.async_copy(rows[k], acc.at[dstbuf.at[i]], ssem[k], add=True)

            @pl.when(nxt < NBLK)
            def _refill():
              pltpu.make_async_copy(rows[k], acc.at[dstbuf.at[0]],
                                    ssem[k]).wait()
              pltpu.make_async_copy(srcoff_hbm.at[h, w, 0], srcb[k],
                                    isem[k]).wait()
              pltpu.async_copy(table_hbm.at[srcb[k]], rows[k], gsem[k])

        return carry

      lax.fori_loop(0, (NBLK + K - 1) // K, body, 0)
      for k in range(K):
        pltpu.make_async_copy(rows[k], acc.at[dstbuf.at[0]], ssem[k]).wait()
      plsc.subcore_barrier()
      pltpu.sync_copy(acc.at[pl.ds(rbase, RS)],
                      out_hbm.at[c, h, pl.ds(rbase, RS)])
      plsc.subcore_barrier()

  return edge_kernel


_edge_pass_1 = _make_edge_pass(1)
_edge_pass_2 = _make_edge_pass(2)


@functools.partial(
    pl.kernel,
    out_type=jax.ShapeDtypeStruct((3 * NW * NP,), jnp.float32),
    mesh=_MESH,
    compiler_params=pltpu.CompilerParams(needs_layout_passes=False),
    scratch_types=[
        pltpu.VMEM((EWP,), jnp.int32),
        pltpu.VMEM((NP + 16,), jnp.float32),
    ],
)
def _deg_kernel(dsts_hbm, out_hbm, dstbuf, hist):
  c = lax.axis_index("c")
  s = lax.axis_index("s")
  w = c * NS + s
  lane = lax.iota(jnp.int32, 16)
  zero16 = jnp.zeros((16,), jnp.float32)
  for m in range(3):
    pltpu.sync_copy(dsts_hbm.at[pl.ds(m * NW * EWP + w * EWP, EWP)], dstbuf)

    def zbody(j, carry):
      hist[pl.ds(j * 16, 16)] = zero16
      return carry

    lax.fori_loop(0, (NP + 16) // 16, zbody, 0)

    def sbody(j, carry):
      d = dstbuf[pl.ds(j * 16, 16)]
      occ, last = plsc.scan_count(d)
      idx = jnp.where(last, d, NP + lane)
      cur = plsc.load_gather(hist, [idx])
      plsc.store_scatter(hist, [idx], cur + occ.astype(jnp.float32))
      return carry

    lax.fori_loop(0, EWP // 16, sbody, 0)
    pltpu.sync_copy(hist.at[pl.ds(0, NP)],
                    out_hbm.at[pl.ds(m * NW * NP + w * NP, NP)])


PB = P // NW
AB = 128


@functools.partial(
    pl.kernel,
    out_type=jax.ShapeDtypeStruct((2, P, AVS), jnp.float32),
    mesh=_MESH,
    scratch_types=[
        pltpu.VMEM((AB,), jnp.int32),
        pltpu.VMEM((AB, AVS), jnp.float32),
        pltpu.SemaphoreType.DMA,
    ],
)
def _action_gather(final_hbm, a01_hbm, out_hbm, idx_v, rows_v, sem):
  c = lax.axis_index("c")
  s = lax.axis_index("s")
  w = c * NS + s
  base = w * PB
  for t in range(2):

    def body(i, carry):
      off = base + i * AB
      pltpu.sync_copy(a01_hbm.at[pl.ds(t * P + off, AB)], idx_v)
      pltpu.async_copy(final_hbm.at[idx_v], rows_v, sem).wait()
      pltpu.sync_copy(rows_v, out_hbm.at[t, pl.ds(off, AB)])
      return carry

    lax.fori_loop(0, PB // AB, body, 0)


def _prep_body(W2l, W2r, b2, W3l, W3r, b3, WlA, blA, WlB, blB, Wout, bout,
               GL, GR, gb, bfin):
  f32 = jnp.float32
  T = jnp.dot(WlB[...], Wout[...], preferred_element_type=f32)
  Whead = jnp.dot(WlA[...], T, preferred_element_type=f32)
  bhead = (jnp.dot(blA[...], T, preferred_element_type=f32)
           + jnp.dot(blB[...], Wout[...], preferred_element_type=f32)
           + bout[...])
  C3 = jnp.dot(W3l[...], Whead, preferred_element_type=f32)
  D3 = jnp.dot(W3r[...], Whead, preferred_element_type=f32)
  CD = jnp.concatenate([C3, D3], axis=1)
  GL[...] = jnp.dot(W2l[...], CD, preferred_element_type=f32)
  GR[...] = jnp.dot(W2r[...], CD, preferred_element_type=f32)
  gb[...] = jnp.dot(b2[...], CD, preferred_element_type=f32)
  bfin[...] = jnp.dot(b3[...], Whead, preferred_element_type=f32) + bhead


NB = 1000


def _degsum(deg_ref):
  return jnp.maximum(jnp.sum(deg_ref[...], axis=1), 1.0)[:, None]


def _combine1_body(parts, deg_ref, obs, W1l, W1r, b1, x1T):
  agg = parts[0, 0] + parts[1, 0]
  mean = agg / _degsum(deg_ref)
  x1 = (jnp.dot(mean, W1l[...], preferred_element_type=jnp.float32)
        + jnp.dot(obs[...], W1r[...], preferred_element_type=jnp.float32)
        + b1[...])
  x1T[0] = x1[:, :F]
  x1T[1] = x1[:, F:]


def _combine2_body(parts, deg_ref, x1T, GL, GR, gb, y3T, z):
  a0 = parts[0, 0] + parts[1, 0]
  a1 = parts[0, 1] + parts[1, 1]
  mean = jnp.concatenate([a0, a1], axis=1) / _degsum(deg_ref)
  x1 = jnp.concatenate([x1T[0], x1T[1]], axis=1)
  yz = (jnp.dot(mean, GL[...], preferred_element_type=jnp.float32)
        + jnp.dot(x1, GR[...], preferred_element_type=jnp.float32)
        + gb[...])
  y3T[...] = yz[:, :F]
  z[...] = yz[:, F:]


def _combine3_body(parts, deg_ref, z, bfin, final):
  agg = parts[0, 0] + parts[1, 0]
  final[...] = agg / _degsum(deg_ref) + z[...] + bfin[...]


def _dotsoft_body(g, out):
  sv = g[0, :, :AVS // 2]
  dv = g[1, :, AVS // 2:]
  logits = jnp.sum(sv * dv, axis=1)
  m = jnp.max(logits)
  e = jnp.exp(logits - m)
  out[0, :] = e / jnp.sum(e)


def kernel(actions, obs, eic, eid, eit, W1l, W1r, b1, W2l, W2r, b2,
           W3l, W3r, b3, WlA, blA, WlB, blB, Wout, bout):
  i32 = jnp.int32
  f32 = jnp.float32
  s_t, d_t = eit[0].astype(i32), eit[1].astype(i32)
  s_c, d_c = eic[0].astype(i32), eic[1].astype(i32)
  s_d, d_d = eid[0].astype(i32), eid[1].astype(i32)
  a01 = jnp.concatenate([actions[..., 0].reshape(-1).astype(i32),
                         actions[..., 1].reshape(-1).astype(i32)])

  def pad_edges(sa, da):
    s2 = sa.reshape(NW, EW)
    d2 = da.reshape(NW, EW)
    padv = jnp.broadcast_to(jnp.arange(EWP - EW, dtype=i32)[None],
                            (NW, EWP - EW))
    s2p = jnp.concatenate([s2, padv % N], axis=1)
    d2p = jnp.concatenate([d2, padv % (NP - N) + N], axis=1)
    return s2p, d2p

  s_tp, d_tp = pad_edges(s_t, d_t)
  s_cp, d_cp = pad_edges(s_c, d_c)
  s_dp, d_dp = pad_edges(s_d, d_d)
  dsts = jnp.concatenate([d_tp.reshape(-1), d_cp.reshape(-1),
                          d_dp.reshape(-1)])
  zeros = jnp.zeros((NP, F), f32)
  b1r = b1.reshape(1, H)
  b2r = b2.reshape(1, H)
  b3r = b3.reshape(1, H)
  blAr = blA.reshape(1, H)
  blBr = blB.reshape(1, H)
  boutr = bout.reshape(1, AVS)

  full = lambda shp: pl.BlockSpec(shp, lambda i: (0,) * len(shp))

  degs = _deg_kernel(dsts).reshape(3, NW, NP).transpose(0, 2, 1)

  GL, GR, gb, bfin = pl.pallas_call(
      _prep_body,
      grid=(1,),
      in_specs=[full((H, H)), full((H, H)), full((1, H)), full((H, H)),
                full((H, H)), full((1, H)), full((H, H)), full((1, H)),
                full((H, H)), full((1, H)), full((H, AVS)), full((1, AVS))],
      out_specs=[full((H, H)), full((H, H)), full((1, H)), full((1, AVS))],
      out_shape=[jax.ShapeDtypeStruct((H, H), f32),
                 jax.ShapeDtypeStruct((H, H), f32),
                 jax.ShapeDtypeStruct((1, H), f32),
                 jax.ShapeDtypeStruct((1, AVS), f32)],
  )(W2l, W2r, b2r, W3l, W3r, b3r, WlA, blAr, WlB, blBr, Wout, boutr)

  p1 = _edge_pass_1(obs, s_tp.reshape(1, NW, NBLK, B),
                    d_tp.reshape(NW, NBLK, B), zeros)

  x1T = pl.pallas_call(
      _combine1_body,
      grid=(N // NB,),
      in_specs=[
          pl.BlockSpec((2, 1, NB, F), lambda i: (0, 0, i, 0)),
          pl.BlockSpec((NB, NW), lambda i: (i, 0)),
          pl.BlockSpec((NB, F), lambda i: (i, 0)),
          full((F, H)), full((F, H)), full((1, H)),
      ],
      out_specs=pl.BlockSpec((2, NB, F), lambda i: (0, i, 0)),
      out_shape=jax.ShapeDtypeStruct((2, N, F), f32),
  )(p1, degs[0], obs, W1l, W1r, b1r)

  srcoff_c = jnp.stack([s_cp, s_cp + N]).reshape(2, NW, NBLK, B)
  p2 = _edge_pass_2(x1T.reshape(2 * N, F), srcoff_c,
                    d_cp.reshape(NW, NBLK, B), zeros)

  y3T, z = pl.pallas_call(
      _combine2_body,
      grid=(N // NB,),
      in_specs=[
          pl.BlockSpec((2, 2, NB, F), lambda i: (0, 0, i, 0)),
          pl.BlockSpec((NB, NW), lambda i: (i, 0)),
          pl.BlockSpec((2, NB, F), lambda i: (0, i, 0)),
          full((H, H)), full((H, H)), full((1, H)),
      ],
      out_specs=[pl.BlockSpec((NB, F), lambda i: (i, 0)),
                 pl.BlockSpec((NB, AVS), lambda i: (i, 0))],
      out_shape=[jax.ShapeDtypeStruct((N, F), f32),
                 jax.ShapeDtypeStruct((N, AVS), f32)],
  )(p2, degs[1], x1T, GL, GR, gb)

  p3 = _edge_pass_1(y3T, s_dp.reshape(1, NW, NBLK, B),
                    d_dp.reshape(NW, NBLK, B), zeros)

  final = pl.pallas_call(
      _combine3_body,
      grid=(N // NB,),
      in_specs=[
          pl.BlockSpec((2, 1, NB, F), lambda i: (0, 0, i, 0)),
          pl.BlockSpec((NB, NW), lambda i: (i, 0)),
          pl.BlockSpec((NB, AVS), lambda i: (i, 0)),
          full((1, AVS)),
      ],
      out_specs=pl.BlockSpec((NB, AVS), lambda i: (i, 0)),
      out_shape=jax.ShapeDtypeStruct((N, AVS), f32),
  )(p3, degs[2], z, bfin)

  g = _action_gather(final, a01)
  probs = pl.pallas_call(
      _dotsoft_body,
      grid=(1,),
      in_specs=[full((2, P, AVS))],
      out_specs=full((1, P)),
      out_shape=jax.ShapeDtypeStruct((1, P), f32),
  )(g)
  return probs


# speedup vs baseline: 1.1293x; 1.0028x over previous
import functools

import jax
import jax.numpy as jnp
from jax import lax
from jax.experimental import pallas as pl
from jax.experimental.pallas import tpu as pltpu
from jax.experimental.pallas import tpu_sc as plsc

N = 10000
E = 320000
F = 128
H = 256
AVS = 128
P = 16384

NC = 2
NS = 16
NW = NC * NS
EW = E // NW
B = 128
NBLK = 80
EWP = NBLK * B
CH = 8
NCH = NBLK // CH
NP = 10112
RS = NP // NS
HH = NP // 2
K = 2

_MESH = plsc.VectorSubcoreMesh(core_axis_name="c", subcore_axis_name="s")


def _make_edge_pass(C):
  @functools.partial(
      pl.kernel,
      out_type=jax.ShapeDtypeStruct((2, C, NP, F), jnp.float32),
      mesh=_MESH,
      scratch_types=(
          [pltpu.VMEM((NBLK, B), jnp.int32)]
          + [pltpu.VMEM((B,), jnp.int32) for _ in range(K)]
          + [pltpu.VMEM((B, F), jnp.float32) for _ in range(K)]
          + [pltpu.SemaphoreType.DMA for _ in range(3 * K)]
          + [pltpu.VMEM_SHARED((NP, F), jnp.float32)]
      ),
  )
  def edge_kernel(table_hbm, srcoff_hbm, dst_hbm, zeros_hbm, out_hbm,
                  dstbuf, *rest):
    srcb = rest[:K]
    rows = rest[K:2 * K]
    gsem = rest[2 * K:3 * K]
    ssem = rest[3 * K:4 * K]
    isem = rest[4 * K:5 * K]
    acc = rest[5 * K]
    c = lax.axis_index("c")
    s = lax.axis_index("s")
    w = c * NS + s
    rbase = s * RS
    pltpu.sync_copy(dst_hbm.at[w], dstbuf)
    for h in range(C):
      pltpu.sync_copy(zeros_hbm.at[pl.ds(rbase, RS)], acc.at[pl.ds(rbase, RS)])
      plsc.subcore_barrier()

      for k in range(K):
        pltpu.sync_copy(srcoff_hbm.at[h, w, k], srcb[k])
        pltpu.async_copy(table_hbm.at[srcb[k]], rows[k], gsem[k])

      def body(j, carry):
        for k in range(K):
          i = j * K + k

          @pl.when(i < NBLK)
          def _process():
            nxt = i + K
            pltpu.make_async_copy(table_hbm.at[srcb[0]], rows[k],
                                  gsem[k]).wait()

            @pl.when(nxt < NBLK)
            def _prefetch_idx():
              pltpu.async_copy(srcoff_hbm.at[h, w, nxt], srcb[k], isem[k])

            pltpu.async_copy(rows[k], acc.at[dstbuf.at[i]], ssem[k], add=True)

            @pl.when(nxt < NBLK)
            def _refill():
              pltpu.make_async_copy(rows[k], acc.at[dstbuf.at[0]],
                                    ssem[k]).wait()
              pltpu.make_async_copy(srcoff_hbm.at[h, w, 0], srcb[k],
                                    isem[k]).wait()
              pltpu.async_copy(table_hbm.at[srcb[k]], rows[k], gsem[k])

        return carry

      lax.fori_loop(0, (NBLK + K - 1) // K, body, 0)
      for k in range(K):
        pltpu.make_async_copy(rows[k], acc.at[dstbuf.at[0]], ssem[k]).wait()
      plsc.subcore_barrier()
      pltpu.sync_copy(acc.at[pl.ds(rbase, RS)],
                      out_hbm.at[c, h, pl.ds(rbase, RS)])
      plsc.subcore_barrier()

  return edge_kernel


_edge_pass_1 = _make_edge_pass(1)
_edge_pass_2 = _make_edge_pass(2)


@functools.partial(
    pl.kernel,
    out_type=jax.ShapeDtypeStruct((3 * NW * NP,), jnp.float32),
    mesh=_MESH,
    compiler_params=pltpu.CompilerParams(needs_layout_passes=False),
    scratch_types=[
        pltpu.VMEM((EWP,), jnp.int32),
        pltpu.VMEM((NP + 16,), jnp.float32),
    ],
)
def _deg_kernel(dsts_hbm, out_hbm, dstbuf, hist):
  c = lax.axis_index("c")
  s = lax.axis_index("s")
  w = c * NS + s
  lane = lax.iota(jnp.int32, 16)
  zero16 = jnp.zeros((16,), jnp.float32)
  for m in range(3):
    pltpu.sync_copy(dsts_hbm.at[pl.ds(m * NW * EWP + w * EWP, EWP)], dstbuf)

    def zbody(j, carry):
      hist[pl.ds(j * 16, 16)] = zero16
      return carry

    lax.fori_loop(0, (NP + 16) // 16, zbody, 0)

    def sbody(j, carry):
      for u in range(4):
        d = dstbuf[pl.ds(j * 64 + u * 16, 16)]
        occ, last = plsc.scan_count(d)
        idx = jnp.where(last, d, NP + lane)
        cur = plsc.load_gather(hist, [idx])
        plsc.store_scatter(hist, [idx], cur + occ.astype(jnp.float32))
      return carry

    lax.fori_loop(0, EWP // 64, sbody, 0)
    pltpu.sync_copy(hist.at[pl.ds(0, NP)],
                    out_hbm.at[pl.ds(m * NW * NP + w * NP, NP)])


PB = P // NW
AB = 128


@functools.partial(
    pl.kernel,
    out_type=jax.ShapeDtypeStruct((2, P, AVS), jnp.float32),
    mesh=_MESH,
    scratch_types=[
        pltpu.VMEM((AB,), jnp.int32),
        pltpu.VMEM((AB, AVS), jnp.float32),
        pltpu.SemaphoreType.DMA,
    ],
)
def _action_gather(final_hbm, a01_hbm, out_hbm, idx_v, rows_v, sem):
  c = lax.axis_index("c")
  s = lax.axis_index("s")
  w = c * NS + s
  base = w * PB
  for t in range(2):

    def body(i, carry):
      off = base + i * AB
      pltpu.sync_copy(a01_hbm.at[pl.ds(t * P + off, AB)], idx_v)
      pltpu.async_copy(final_hbm.at[idx_v], rows_v, sem).wait()
      pltpu.sync_copy(rows_v, out_hbm.at[t, pl.ds(off, AB)])
      return carry

    lax.fori_loop(0, PB // AB, body, 0)


def _prep_body(W2l, W2r, b2, W3l, W3r, b3, WlA, blA, WlB, blB, Wout, bout,
               GL, GR, gb, bfin):
  f32 = jnp.float32
  T = jnp.dot(WlB[...], Wout[...], preferred_element_type=f32)
  Whead = jnp.dot(WlA[...], T, preferred_element_type=f32)
  bhead = (jnp.dot(blA[...], T, preferred_element_type=f32)
           + jnp.dot(blB[...], Wout[...], preferred_element_type=f32)
           + bout[...])
  C3 = jnp.dot(W3l[...], Whead, preferred_element_type=f32)
  D3 = jnp.dot(W3r[...], Whead, preferred_element_type=f32)
  CD = jnp.concatenate([C3, D3], axis=1)
  GL[...] = jnp.dot(W2l[...], CD, preferred_element_type=f32)
  GR[...] = jnp.dot(W2r[...], CD, preferred_element_type=f32)
  gb[...] = jnp.dot(b2[...], CD, preferred_element_type=f32)
  bfin[...] = jnp.dot(b3[...], Whead, preferred_element_type=f32) + bhead


NB = 1000


def _degsum(deg_ref):
  return jnp.maximum(jnp.sum(deg_ref[...], axis=1), 1.0)[:, None]


def _combine1_body(parts, deg_ref, obs, W1l, W1r, b1, x1T):
  agg = parts[0, 0] + parts[1, 0]
  mean = agg / _degsum(deg_ref)
  x1 = (jnp.dot(mean, W1l[...], preferred_element_type=jnp.float32)
        + jnp.dot(obs[...], W1r[...], preferred_element_type=jnp.float32)
        + b1[...])
  x1T[0] = x1[:, :F]
  x1T[1] = x1[:, F:]


def _combine2_body(parts, deg_ref, x1T, GL, GR, gb, y3T, z):
  a0 = parts[0, 0] + parts[1, 0]
  a1 = parts[0, 1] + parts[1, 1]
  mean = jnp.concatenate([a0, a1], axis=1) / _degsum(deg_ref)
  x1 = jnp.concatenate([x1T[0], x1T[1]], axis=1)
  yz = (jnp.dot(mean, GL[...], preferred_element_type=jnp.float32)
        + jnp.dot(x1, GR[...], preferred_element_type=jnp.float32)
        + gb[...])
  y3T[...] = yz[:, :F]
  z[...] = yz[:, F:]


def _combine3_body(parts, deg_ref, z, bfin, final):
  agg = parts[0, 0] + parts[1, 0]
  final[...] = agg / _degsum(deg_ref) + z[...] + bfin[...]


def _dotsoft_body(g, out):
  sv = g[0, :, :AVS // 2]
  dv = g[1, :, AVS // 2:]
  logits = jnp.sum(sv * dv, axis=1)
  m = jnp.max(logits)
  e = jnp.exp(logits - m)
  out[0, :] = e / jnp.sum(e)


def kernel(actions, obs, eic, eid, eit, W1l, W1r, b1, W2l, W2r, b2,
           W3l, W3r, b3, WlA, blA, WlB, blB, Wout, bout):
  i32 = jnp.int32
  f32 = jnp.float32
  s_t, d_t = eit[0].astype(i32), eit[1].astype(i32)
  s_c, d_c = eic[0].astype(i32), eic[1].astype(i32)
  s_d, d_d = eid[0].astype(i32), eid[1].astype(i32)
  a01 = jnp.concatenate([actions[..., 0].reshape(-1).astype(i32),
                         actions[..., 1].reshape(-1).astype(i32)])

  def pad_edges(sa, da):
    s2 = sa.reshape(NW, EW)
    d2 = da.reshape(NW, EW)
    padv = jnp.broadcast_to(jnp.arange(EWP - EW, dtype=i32)[None],
                            (NW, EWP - EW))
    s2p = jnp.concatenate([s2, padv % N], axis=1)
    d2p = jnp.concatenate([d2, padv % (NP - N) + N], axis=1)
    return s2p, d2p

  s_tp, d_tp = pad_edges(s_t, d_t)
  s_cp, d_cp = pad_edges(s_c, d_c)
  s_dp, d_dp = pad_edges(s_d, d_d)
  dsts = jnp.concatenate([d_tp.reshape(-1), d_cp.reshape(-1),
                          d_dp.reshape(-1)])
  zeros = jnp.zeros((NP, F), f32)
  b1r = b1.reshape(1, H)
  b2r = b2.reshape(1, H)
  b3r = b3.reshape(1, H)
  blAr = blA.reshape(1, H)
  blBr = blB.reshape(1, H)
  boutr = bout.reshape(1, AVS)

  full = lambda shp: pl.BlockSpec(shp, lambda i: (0,) * len(shp))

  degs = _deg_kernel(dsts).reshape(3, NW, NP).transpose(0, 2, 1)

  GL, GR, gb, bfin = pl.pallas_call(
      _prep_body,
      grid=(1,),
      in_specs=[full((H, H)), full((H, H)), full((1, H)), full((H, H)),
                full((H, H)), full((1, H)), full((H, H)), full((1, H)),
                full((H, H)), full((1, H)), full((H, AVS)), full((1, AVS))],
      out_specs=[full((H, H)), full((H, H)), full((1, H)), full((1, AVS))],
      out_shape=[jax.ShapeDtypeStruct((H, H), f32),
                 jax.ShapeDtypeStruct((H, H), f32),
                 jax.ShapeDtypeStruct((1, H), f32),
                 jax.ShapeDtypeStruct((1, AVS), f32)],
  )(W2l, W2r, b2r, W3l, W3r, b3r, WlA, blAr, WlB, blBr, Wout, boutr)

  p1 = _edge_pass_1(obs, s_tp.reshape(1, NW, NBLK, B),
                    d_tp.reshape(NW, NBLK, B), zeros)

  x1T = pl.pallas_call(
      _combine1_body,
      grid=(N // NB,),
      in_specs=[
          pl.BlockSpec((2, 1, NB, F), lambda i: (0, 0, i, 0)),
          pl.BlockSpec((NB, NW), lambda i: (i, 0)),
          pl.BlockSpec((NB, F), lambda i: (i, 0)),
          full((F, H)), full((F, H)), full((1, H)),
      ],
      out_specs=pl.BlockSpec((2, NB, F), lambda i: (0, i, 0)),
      out_shape=jax.ShapeDtypeStruct((2, N, F), f32),
  )(p1, degs[0], obs, W1l, W1r, b1r)

  srcoff_c = jnp.stack([s_cp, s_cp + N]).reshape(2, NW, NBLK, B)
  p2 = _edge_pass_2(x1T.reshape(2 * N, F), srcoff_c,
                    d_cp.reshape(NW, NBLK, B), zeros)

  y3T, z = pl.pallas_call(
      _combine2_body,
      grid=(N // NB,),
      in_specs=[
          pl.BlockSpec((2, 2, NB, F), lambda i: (0, 0, i, 0)),
          pl.BlockSpec((NB, NW), lambda i: (i, 0)),
          pl.BlockSpec((2, NB, F), lambda i: (0, i, 0)),
          full((H, H)), full((H, H)), full((1, H)),
      ],
      out_specs=[pl.BlockSpec((NB, F), lambda i: (i, 0)),
                 pl.BlockSpec((NB, AVS), lambda i: (i, 0))],
      out_shape=[jax.ShapeDtypeStruct((N, F), f32),
                 jax.ShapeDtypeStruct((N, AVS), f32)],
  )(p2, degs[1], x1T, GL, GR, gb)

  p3 = _edge_pass_1(y3T, s_dp.reshape(1, NW, NBLK, B),
                    d_dp.reshape(NW, NBLK, B), zeros)

  final = pl.pallas_call(
      _combine3_body,
      grid=(N // NB,),
      in_specs=[
          pl.BlockSpec((2, 1, NB, F), lambda i: (0, 0, i, 0)),
          pl.BlockSpec((NB, NW), lambda i: (i, 0)),
          pl.BlockSpec((NB, AVS), lambda i: (i, 0)),
          full((1, AVS)),
      ],
      out_specs=pl.BlockSpec((NB, AVS), lambda i: (i, 0)),
      out_shape=jax.ShapeDtypeStruct((N, AVS), f32),
  )(p3, degs[2], z, bfin)

  g = _action_gather(final, a01)
  probs = pl.pallas_call(
      _dotsoft_body,
      grid=(1,),
      in_specs=[full((2, P, AVS))],
      out_specs=full((1, P)),
      out_shape=jax.ShapeDtypeStruct((1, P), f32),
  )(g)
  return probs


# speedup vs baseline: 1.1658x; 1.0323x over previous
import functools

import jax
import jax.numpy as jnp
from jax import lax
from jax.experimental import pallas as pl
from jax.experimental.pallas import tpu as pltpu
from jax.experimental.pallas import tpu_sc as plsc

N = 10000
E = 320000
F = 128
H = 256
AVS = 128
P = 16384

NC = 2
NS = 16
NW = NC * NS
EW = E // NW
B = 128
NBLK = 80
EWP = NBLK * B
CH = 8
NCH = NBLK // CH
NP = 10112
RS = NP // NS
HH = NP // 2
K = 2

_MESH = plsc.VectorSubcoreMesh(core_axis_name="c", subcore_axis_name="s")


def _make_edge_pass(C):
  @functools.partial(
      pl.kernel,
      out_type=(jax.ShapeDtypeStruct((2, C, NP, F), jnp.float32),
                jax.ShapeDtypeStruct((NW * NP,), jnp.float32)),
      mesh=_MESH,
      compiler_params=pltpu.CompilerParams(needs_layout_passes=False),
      scratch_types=(
          [pltpu.VMEM((B,), jnp.int32) for _ in range(2 * K)]
          + [pltpu.VMEM((B, F), jnp.float32) for _ in range(K)]
          + [pltpu.VMEM((NP + 16,), jnp.float32)]
          + [pltpu.SemaphoreType.DMA for _ in range(4 * K)]
          + [pltpu.VMEM_SHARED((NP, F), jnp.float32)]
      ),
  )
  def edge_kernel(table_hbm, srcoff_hbm, dst_hbm, zeros_hbm, out_hbm,
                  degout_hbm, *rest):
    srcb = rest[:K]
    dstb = rest[K:2 * K]
    rows = rest[2 * K:3 * K]
    hist = rest[3 * K]
    sems = rest[3 * K + 1:3 * K + 1 + 4 * K]
    gsem = sems[:K]
    ssem = sems[K:2 * K]
    isem = sems[2 * K:3 * K]
    dsem = sems[3 * K:4 * K]
    acc = rest[3 * K + 1 + 4 * K]
    c = lax.axis_index("c")
    s = lax.axis_index("s")
    w = c * NS + s
    rbase = s * RS
    lane = lax.iota(jnp.int32, 16)
    zero16 = jnp.zeros((16,), jnp.float32)

    def zbody(j, carry):
      hist[pl.ds(j * 16, 16)] = zero16
      return carry

    lax.fori_loop(0, (NP + 16) // 16, zbody, 0)
    for h in range(C):
      pltpu.sync_copy(zeros_hbm.at[pl.ds(rbase, RS)], acc.at[pl.ds(rbase, RS)])
      plsc.subcore_barrier()

      for k in range(K):
        pltpu.sync_copy(srcoff_hbm.at[h, w, k], srcb[k])
        pltpu.sync_copy(dst_hbm.at[w, k], dstb[k])
        pltpu.async_copy(table_hbm.at[srcb[k]], rows[k], gsem[k])

      def body(j, carry):
        for k in range(K):
          i = j * K + k

          @pl.when(i < NBLK)
          def _process():
            nxt = i + K
            pltpu.make_async_copy(table_hbm.at[srcb[0]], rows[k],
                                  gsem[k]).wait()

            @pl.when(nxt < NBLK)
            def _prefetch_src():
              pltpu.async_copy(srcoff_hbm.at[h, w, nxt], srcb[k], isem[k])

            @pl.when(i >= K)
            def _wait_dst():
              pltpu.make_async_copy(dst_hbm.at[w, 0], dstb[k], dsem[k]).wait()

            pltpu.async_copy(rows[k], acc.at[dstb[k]], ssem[k], add=True)

            if h == 0:
              for u in range(B // 16):
                d = dstb[k][pl.ds(u * 16, 16)]
                occ, last = plsc.scan_count(d)
                idx = jnp.where(last, d, NP + lane)
                cur = plsc.load_gather(hist, [idx])
                plsc.store_scatter(hist, [idx], cur + occ.astype(jnp.float32))

            @pl.when(nxt < NBLK)
            def _refill():
              pltpu.make_async_copy(rows[k], acc.at[dstb[0]], ssem[k]).wait()
              pltpu.async_copy(dst_hbm.at[w, nxt], dstb[k], dsem[k])
              pltpu.make_async_copy(srcoff_hbm.at[h, w, 0], srcb[k],
                                    isem[k]).wait()
              pltpu.async_copy(table_hbm.at[srcb[k]], rows[k], gsem[k])

        return carry

      lax.fori_loop(0, (NBLK + K - 1) // K, body, 0)
      for k in range(K):
        pltpu.make_async_copy(rows[k], acc.at[dstb[0]], ssem[k]).wait()
      if h == 0:
        pltpu.sync_copy(hist.at[pl.ds(0, NP)],
                        degout_hbm.at[pl.ds(w * NP, NP)])
      plsc.subcore_barrier()
      pltpu.sync_copy(acc.at[pl.ds(rbase, RS)],
                      out_hbm.at[c, h, pl.ds(rbase, RS)])
      plsc.subcore_barrier()

  return edge_kernel


_edge_pass_1 = _make_edge_pass(1)
_edge_pass_2 = _make_edge_pass(2)


PB = P // NW
AB = 128


@functools.partial(
    pl.kernel,
    out_type=jax.ShapeDtypeStruct((2, P, AVS), jnp.float32),
    mesh=_MESH,
    scratch_types=[
        pltpu.VMEM((AB,), jnp.int32),
        pltpu.VMEM((AB, AVS), jnp.float32),
        pltpu.SemaphoreType.DMA,
    ],
)
def _action_gather(final_hbm, a01_hbm, out_hbm, idx_v, rows_v, sem):
  c = lax.axis_index("c")
  s = lax.axis_index("s")
  w = c * NS + s
  base = w * PB
  for t in range(2):

    def body(i, carry):
      off = base + i * AB
      pltpu.sync_copy(a01_hbm.at[pl.ds(t * P + off, AB)], idx_v)
      pltpu.async_copy(final_hbm.at[idx_v], rows_v, sem).wait()
      pltpu.sync_copy(rows_v, out_hbm.at[t, pl.ds(off, AB)])
      return carry

    lax.fori_loop(0, PB // AB, body, 0)


def _prep_body(W2l, W2r, b2, W3l, W3r, b3, WlA, blA, WlB, blB, Wout, bout,
               GL, GR, gb, bfin):
  f32 = jnp.float32
  T = jnp.dot(WlB[...], Wout[...], preferred_element_type=f32)
  Whead = jnp.dot(WlA[...], T, preferred_element_type=f32)
  bhead = (jnp.dot(blA[...], T, preferred_element_type=f32)
           + jnp.dot(blB[...], Wout[...], preferred_element_type=f32)
           + bout[...])
  C3 = jnp.dot(W3l[...], Whead, preferred_element_type=f32)
  D3 = jnp.dot(W3r[...], Whead, preferred_element_type=f32)
  CD = jnp.concatenate([C3, D3], axis=1)
  GL[...] = jnp.dot(W2l[...], CD, preferred_element_type=f32)
  GR[...] = jnp.dot(W2r[...], CD, preferred_element_type=f32)
  gb[...] = jnp.dot(b2[...], CD, preferred_element_type=f32)
  bfin[...] = jnp.dot(b3[...], Whead, preferred_element_type=f32) + bhead


NB = 1000


def _degsum(deg_ref):
  return jnp.maximum(jnp.sum(deg_ref[...], axis=1), 1.0)[:, None]


def _combine1_body(parts, deg_ref, obs, W1l, W1r, b1, x1T):
  agg = parts[0, 0] + parts[1, 0]
  mean = agg / _degsum(deg_ref)
  x1 = (jnp.dot(mean, W1l[...], preferred_element_type=jnp.float32)
        + jnp.dot(obs[...], W1r[...], preferred_element_type=jnp.float32)
        + b1[...])
  x1T[0] = x1[:, :F]
  x1T[1] = x1[:, F:]


def _combine2_body(parts, deg_ref, x1T, GL, GR, gb, y3T, z):
  a0 = parts[0, 0] + parts[1, 0]
  a1 = parts[0, 1] + parts[1, 1]
  mean = jnp.concatenate([a0, a1], axis=1) / _degsum(deg_ref)
  x1 = jnp.concatenate([x1T[0], x1T[1]], axis=1)
  yz = (jnp.dot(mean, GL[...], preferred_element_type=jnp.float32)
        + jnp.dot(x1, GR[...], preferred_element_type=jnp.float32)
        + gb[...])
  y3T[...] = yz[:, :F]
  z[...] = yz[:, F:]


def _combine3_body(parts, deg_ref, z, bfin, final):
  agg = parts[0, 0] + parts[1, 0]
  final[...] = agg / _degsum(deg_ref) + z[...] + bfin[...]


def _dotsoft_body(g, out):
  sv = g[0, :, :AVS // 2]
  dv = g[1, :, AVS // 2:]
  logits = jnp.sum(sv * dv, axis=1)
  m = jnp.max(logits)
  e = jnp.exp(logits - m)
  out[0, :] = e / jnp.sum(e)


def kernel(actions, obs, eic, eid, eit, W1l, W1r, b1, W2l, W2r, b2,
           W3l, W3r, b3, WlA, blA, WlB, blB, Wout, bout):
  i32 = jnp.int32
  f32 = jnp.float32
  s_t, d_t = eit[0].astype(i32), eit[1].astype(i32)
  s_c, d_c = eic[0].astype(i32), eic[1].astype(i32)
  s_d, d_d = eid[0].astype(i32), eid[1].astype(i32)
  a01 = jnp.concatenate([actions[..., 0].reshape(-1).astype(i32),
                         actions[..., 1].reshape(-1).astype(i32)])

  def pad_edges(sa, da):
    s2 = sa.reshape(NW, EW)
    d2 = da.reshape(NW, EW)
    padv = jnp.broadcast_to(jnp.arange(EWP - EW, dtype=i32)[None],
                            (NW, EWP - EW))
    s2p = jnp.concatenate([s2, padv % N], axis=1)
    d2p = jnp.concatenate([d2, padv % (NP - N) + N], axis=1)
    return s2p, d2p

  s_tp, d_tp = pad_edges(s_t, d_t)
  s_cp, d_cp = pad_edges(s_c, d_c)
  s_dp, d_dp = pad_edges(s_d, d_d)
  zeros = jnp.zeros((NP, F), f32)
  b1r = b1.reshape(1, H)
  b2r = b2.reshape(1, H)
  b3r = b3.reshape(1, H)
  blAr = blA.reshape(1, H)
  blBr = blB.reshape(1, H)
  boutr = bout.reshape(1, AVS)

  full = lambda shp: pl.BlockSpec(shp, lambda i: (0,) * len(shp))

  GL, GR, gb, bfin = pl.pallas_call(
      _prep_body,
      grid=(1,),
      in_specs=[full((H, H)), full((H, H)), full((1, H)), full((H, H)),
                full((H, H)), full((1, H)), full((H, H)), full((1, H)),
                full((H, H)), full((1, H)), full((H, AVS)), full((1, AVS))],
      out_specs=[full((H, H)), full((H, H)), full((1, H)), full((1, AVS))],
      out_shape=[jax.ShapeDtypeStruct((H, H), f32),
                 jax.ShapeDtypeStruct((H, H), f32),
                 jax.ShapeDtypeStruct((1, H), f32),
                 jax.ShapeDtypeStruct((1, AVS), f32)],
  )(W2l, W2r, b2r, W3l, W3r, b3r, WlA, blAr, WlB, blBr, Wout, boutr)

  p1, deg_t = _edge_pass_1(obs, s_tp.reshape(1, NW, NBLK, B),
                           d_tp.reshape(NW, NBLK, B), zeros)
  deg_t = deg_t.reshape(NW, NP).T

  x1T = pl.pallas_call(
      _combine1_body,
      grid=(N // NB,),
      in_specs=[
          pl.BlockSpec((2, 1, NB, F), lambda i: (0, 0, i, 0)),
          pl.BlockSpec((NB, NW), lambda i: (i, 0)),
          pl.BlockSpec((NB, F), lambda i: (i, 0)),
          full((F, H)), full((F, H)), full((1, H)),
      ],
      out_specs=pl.BlockSpec((2, NB, F), lambda i: (0, i, 0)),
      out_shape=jax.ShapeDtypeStruct((2, N, F), f32),
  )(p1, deg_t, obs, W1l, W1r, b1r)

  srcoff_c = jnp.stack([s_cp, s_cp + N]).reshape(2, NW, NBLK, B)
  p2, deg_c = _edge_pass_2(x1T.reshape(2 * N, F), srcoff_c,
                           d_cp.reshape(NW, NBLK, B), zeros)
  deg_c = deg_c.reshape(NW, NP).T

  y3T, z = pl.pallas_call(
      _combine2_body,
      grid=(N // NB,),
      in_specs=[
          pl.BlockSpec((2, 2, NB, F), lambda i: (0, 0, i, 0)),
          pl.BlockSpec((NB, NW), lambda i: (i, 0)),
          pl.BlockSpec((2, NB, F), lambda i: (0, i, 0)),
          full((H, H)), full((H, H)), full((1, H)),
      ],
      out_specs=[pl.BlockSpec((NB, F), lambda i: (i, 0)),
                 pl.BlockSpec((NB, AVS), lambda i: (i, 0))],
      out_shape=[jax.ShapeDtypeStruct((N, F), f32),
                 jax.ShapeDtypeStruct((N, AVS), f32)],
  )(p2, deg_c, x1T, GL, GR, gb)

  p3, deg_d = _edge_pass_1(y3T, s_dp.reshape(1, NW, NBLK, B),
                           d_dp.reshape(NW, NBLK, B), zeros)
  deg_d = deg_d.reshape(NW, NP).T

  final = pl.pallas_call(
      _combine3_body,
      grid=(N // NB,),
      in_specs=[
          pl.BlockSpec((2, 1, NB, F), lambda i: (0, 0, i, 0)),
          pl.BlockSpec((NB, NW), lambda i: (i, 0)),
          pl.BlockSpec((NB, AVS), lambda i: (i, 0)),
          full((1, AVS)),
      ],
      out_specs=pl.BlockSpec((NB, AVS), lambda i: (i, 0)),
      out_shape=jax.ShapeDtypeStruct((N, AVS), f32),
  )(p3, deg_d, z, bfin)

  g = _action_gather(final, a01)
  probs = pl.pallas_call(
      _dotsoft_body,
      grid=(1,),
      in_specs=[full((2, P, AVS))],
      out_specs=full((1, P)),
      out_shape=jax.ShapeDtypeStruct((1, P), f32),
  )(g)
  return probs
